# baseline (speedup 1.0000x reference)
import jax
import jax.numpy as jnp
from jax import lax
from jax.experimental import pallas as pl
from jax.experimental.pallas import tpu as pltpu
from jax.experimental.pallas import tpu_sc as plsc

N = 10000
E = 320000
D = 128

NC = 2
NS = 16
NW = NC * NS

CH = 64
NCH = 160
EPW = NCH * CH
EPAD = NW * EPW

CHA = 64
NBUF = 2
NCHA = 162
EPWA = NCHA * CHA
EPADA = NW * EPWA

_SHIFT = 14
_MASK = (1 << _SHIFT) - 1

_STRIDE = 624
_ROWS = 640

_mesh = plsc.VectorSubcoreMesh(core_axis_name="c", subcore_axis_name="s")


def _sc_deg_body(pk_hbm, w_hbm, out_hbm, pm, wm, dstm, dbuf, sem, deg_sh):
    cid = lax.axis_index("c")
    sid = lax.axis_index("s")
    wid = sid * NC + cid
    roff = sid * _STRIDE

    for t in range(_ROWS // 16):
        dbuf[pl.ds(t * 16, 16)] = jnp.zeros((16,), jnp.float32)
    pltpu.sync_copy(dbuf, deg_sh.at[pl.ds(roff, _ROWS)])

    pltpu.sync_copy(pk_hbm.at[pl.ds(wid * NCH, NCH)], pm)
    pltpu.sync_copy(w_hbm.at[pl.ds(wid * NCH, NCH)], wm)

    def unpack(j, carry):
        for t in range(CH // 16):
            sl = pl.ds(t * 16, 16)
            dstm[j, sl] = jnp.bitwise_and(pm[j, sl], _MASK)
        return carry

    lax.fori_loop(0, NCH, unpack, 0)
    plsc.subcore_barrier()

    def group(gg, carry):
        for t in range(8):
            j = gg * 8 + t
            pltpu.async_copy(wm.at[j], deg_sh.at[dstm.at[j]], sem, add=True)
        for t in range(8):
            j = gg * 8 + t
            pltpu.make_async_copy(wm.at[j], deg_sh.at[dstm.at[j]], sem).wait()
        return carry

    lax.fori_loop(0, NCH // 8, group, 0)
    plsc.subcore_barrier()

    pltpu.sync_copy(deg_sh.at[pl.ds(roff, _ROWS)], dbuf)
    pltpu.sync_copy(dbuf, out_hbm.at[pl.ds(cid * N + roff, _ROWS)])


_sc_deg = pl.kernel(
    _sc_deg_body,
    out_type=jax.ShapeDtypeStruct((NC * N,), jnp.float32),
    mesh=_mesh,
    scratch_types=[
        pltpu.VMEM((NCH, CH), jnp.int32),
        pltpu.VMEM((NCH, CH), jnp.float32),
        pltpu.VMEM((NCH, CH), jnp.int32),
        pltpu.VMEM((_ROWS,), jnp.float32),
        pltpu.SemaphoreType.DMA,
        pltpu.VMEM_SHARED((N,), jnp.float32),
    ],
)


def _sc_agg_body(g_hbm, pk_hbm, w_hbm, out_hbm, *refs):
    pkv = refs[0:NBUF]
    wv = refs[NBUF:2 * NBUF]
    srcv = refs[2 * NBUF:3 * NBUF]
    dstv = refs[3 * NBUF:4 * NBUF]
    rows = refs[4 * NBUF:5 * NBUF]
    gsem = refs[5 * NBUF:6 * NBUF]
    isem = refs[6 * NBUF:7 * NBUF]
    ssem = refs[7 * NBUF:8 * NBUF]
    acc_sh = refs[8 * NBUF]

    cid = lax.axis_index("c")
    sid = lax.axis_index("s")
    wid = sid * NC + cid
    roff = sid * _STRIDE
    ebase = wid * EPWA

    def zrow(r, carry):
        for j in range(8):
            rows[0][r, pl.ds(j * 16, 16)] = jnp.zeros((16,), jnp.float32)
        return carry

    lax.fori_loop(0, CHA, zrow, 0)
    for t in range(_ROWS // 40):
        pltpu.sync_copy(rows[0].at[pl.ds(0, 40)],
                        acc_sh.at[pl.ds(roff + t * 40, 40)])
    plsc.subcore_barrier()

    def issue_idx(j, b):
        off = ebase + j * CHA
        pltpu.async_copy(pk_hbm.at[pl.ds(off, CHA)], pkv[b], isem[b])
        pltpu.async_copy(w_hbm.at[pl.ds(off, CHA)], wv[b], isem[b])

    def wait_idx(b):
        pltpu.make_async_copy(pk_hbm.at[pl.ds(0, CHA)], pkv[b], isem[b]).wait()
        pltpu.make_async_copy(w_hbm.at[pl.ds(0, CHA)], wv[b], isem[b]).wait()

    def unpack(b):
        for t in range(CHA // 16):
            sl = pl.ds(t * 16, 16)
            p16 = pkv[b][sl]
            dstv[b][sl] = jnp.bitwise_and(p16, _MASK)
            srcv[b][sl] = lax.shift_right_logical(p16, _SHIFT)

    def issue_gather(b):
        pltpu.async_copy(g_hbm.at[srcv[b]], rows[b], gsem[b])

    def wait_gather(b):
        pltpu.make_async_copy(g_hbm.at[pl.ds(0, CHA)], rows[b], gsem[b]).wait()

    def issue_scatter(b):
        pltpu.async_copy(rows[b], acc_sh.at[dstv[b]], ssem[b], add=True)

    def wait_scatter(b):
        pltpu.make_async_copy(rows[b], acc_sh.at[dstv[b]], ssem[b]).wait()

    def scale(b):
        def egroup(gi, c2):
            w16 = wv[b][pl.ds(gi * 16, 16)]
            for e in range(16):
                sp = jnp.broadcast_to(lax.slice(w16, (e,), (e + 1,)), (16,))
                r = gi * 16 + e
                for jj in range(8):
                    sl = pl.ds(jj * 16, 16)
                    rows[b][r, sl] = rows[b][r, sl] * sp
            return c2
        lax.fori_loop(0, CHA // 16, egroup, 0)

    for b in range(NBUF):
        issue_idx(b, b)
    for b in range(2):
        wait_idx(b)
        unpack(b)
        issue_gather(b)

    def slot(j, b):
        wait_gather(b)

        @pl.when(j + NBUF < NCHA)
        def _():
            issue_idx(j + NBUF, b)

        issue_scatter(b)

        b2 = (b + 2) % NBUF

        @pl.when(j + 2 < NCHA)
        def _():
            @pl.when(j >= NBUF - 2)
            def _():
                wait_scatter(b2)
            wait_idx(b2)
            unpack(b2)
            issue_gather(b2)

    def quad(gq, carry):
        j0 = gq * NBUF
        for t in range(NBUF):
            slot(j0 + t, t)
        return carry

    lax.fori_loop(0, NCHA // NBUF, quad, 0)
    wait_scatter((NCHA - 2) % NBUF)
    wait_scatter((NCHA - 1) % NBUF)
    plsc.subcore_barrier()

    for t in range(_ROWS // 40):
        pltpu.sync_copy(acc_sh.at[pl.ds(roff + t * 40, 40)],
                        rows[0].at[pl.ds(0, 40)])
        pltpu.sync_copy(rows[0].at[pl.ds(0, 40)],
                        out_hbm.at[cid, pl.ds(roff + t * 40, 40)])


_sc_agg = pl.kernel(
    _sc_agg_body,
    out_type=jax.ShapeDtypeStruct((NC, N, D), jnp.float32),
    mesh=_mesh,
    scratch_types=(
        [pltpu.VMEM((CHA,), jnp.int32) for _ in range(NBUF)] +
        [pltpu.VMEM((CHA,), jnp.float32) for _ in range(NBUF)] +
        [pltpu.VMEM((CHA,), jnp.int32) for _ in range(NBUF)] +
        [pltpu.VMEM((CHA,), jnp.int32) for _ in range(NBUF)] +
        [pltpu.VMEM((CHA, D), jnp.float32) for _ in range(NBUF)] +
        [pltpu.SemaphoreType.DMA for _ in range(3 * NBUF)] +
        [pltpu.VMEM_SHARED((N, D), jnp.float32)]
    ),
)


def _tc_pre_body(x_ref, w_ref, deg_ref, g_ref, dis_ref):
    h = jnp.dot(x_ref[...], w_ref[...], preferred_element_type=jnp.float32)
    deg = deg_ref[0, :, 0] + deg_ref[1, :, 0] + 1.0
    dis = lax.rsqrt(deg)
    g_ref[...] = h * dis[:, None]
    dis_ref[...] = dis[:, None]


_BR = 1000

_tc_pre = pl.pallas_call(
    _tc_pre_body,
    grid=(N // _BR,),
    in_specs=[
        pl.BlockSpec((_BR, D), lambda i: (i, 0)),
        pl.BlockSpec((D, D), lambda i: (0, 0)),
        pl.BlockSpec((NC, _BR, 1), lambda i: (0, i, 0)),
    ],
    out_specs=[
        pl.BlockSpec((_BR, D), lambda i: (i, 0)),
        pl.BlockSpec((_BR, 1), lambda i: (i, 0)),
    ],
    out_shape=[
        jax.ShapeDtypeStruct((N, D), jnp.float32),
        jax.ShapeDtypeStruct((N, 1), jnp.float32),
    ],
)


def _tc_post_body(acc_ref, g_ref, dis_ref, b_ref, o_ref):
    s = acc_ref[0] + acc_ref[1] + g_ref[...]
    o_ref[...] = jnp.maximum(s * dis_ref[...] + b_ref[...], 0.0)


_tc_post = pl.pallas_call(
    _tc_post_body,
    grid=(N // _BR,),
    in_specs=[
        pl.BlockSpec((NC, _BR, D), lambda i: (0, i, 0)),
        pl.BlockSpec((_BR, D), lambda i: (i, 0)),
        pl.BlockSpec((_BR, 1), lambda i: (i, 0)),
        pl.BlockSpec((1, D), lambda i: (0, 0)),
    ],
    out_specs=pl.BlockSpec((_BR, D), lambda i: (i, 0)),
    out_shape=jax.ShapeDtypeStruct((N, D), jnp.float32),
)


def kernel(x, edge_index, edge_weights, W, b):
    src = edge_index[0]
    dst = edge_index[1]
    packed = src * (1 << _SHIFT) + dst
    pad2 = lambda a: jnp.pad(a, (0, EPAD - E)).reshape(NW * NCH, CH)
    pad1 = lambda a: jnp.pad(a, (0, EPADA - E))
    deg2 = _sc_deg(pad2(packed), pad2(edge_weights))
    g, dis = _tc_pre(x, W, deg2.reshape(NC, N, 1))
    acc2 = _sc_agg(g, pad1(packed), pad1(edge_weights))
    return _tc_post(acc2, g, dis, b.reshape(1, D))


# speedup vs baseline: 1.2084x; 1.2084x over previous
import jax
import jax.numpy as jnp
from jax import lax
from jax.experimental import pallas as pl
from jax.experimental.pallas import tpu as pltpu
from jax.experimental.pallas import tpu_sc as plsc

N = 10000
E = 320000
D = 128

NC = 2
NS = 16
NW = NC * NS

CH = 64
NCH = 160
EPW = NCH * CH
EPAD = NW * EPW

CHA = 64
NBUF = 2
NCH0 = 94
NCH1 = 226
NCHT = NCH0 + NCH1
EPADA = NS * NCHT * CHA

_SHIFT = 14
_MASK = (1 << _SHIFT) - 1

_STRIDE = 624
_ROWS = 640

_mesh = plsc.VectorSubcoreMesh(core_axis_name="c", subcore_axis_name="s")


def _sc_deg_body(pk_hbm, w_hbm, out_hbm, pm, wm, dstm, dbuf, sem, deg_sh):
    cid = lax.axis_index("c")
    sid = lax.axis_index("s")
    wid = sid * NC + cid
    roff = sid * _STRIDE

    for t in range(_ROWS // 16):
        dbuf[pl.ds(t * 16, 16)] = jnp.zeros((16,), jnp.float32)
    pltpu.sync_copy(dbuf, deg_sh.at[pl.ds(roff, _ROWS)])

    pltpu.sync_copy(pk_hbm.at[pl.ds(wid * NCH, NCH)], pm)
    pltpu.sync_copy(w_hbm.at[pl.ds(wid * NCH, NCH)], wm)

    def unpack(j, carry):
        for t in range(CH // 16):
            sl = pl.ds(t * 16, 16)
            dstm[j, sl] = jnp.bitwise_and(pm[j, sl], _MASK)
        return carry

    lax.fori_loop(0, NCH, unpack, 0)
    plsc.subcore_barrier()

    def group(gg, carry):
        for t in range(8):
            j = gg * 8 + t
            pltpu.async_copy(wm.at[j], deg_sh.at[dstm.at[j]], sem, add=True)
        for t in range(8):
            j = gg * 8 + t
            pltpu.make_async_copy(wm.at[j], deg_sh.at[dstm.at[j]], sem).wait()
        return carry

    lax.fori_loop(0, NCH // 8, group, 0)
    plsc.subcore_barrier()

    pltpu.sync_copy(deg_sh.at[pl.ds(roff, _ROWS)], dbuf)
    pltpu.sync_copy(dbuf, out_hbm.at[pl.ds(cid * N + roff, _ROWS)])


_sc_deg = pl.kernel(
    _sc_deg_body,
    out_type=jax.ShapeDtypeStruct((NC * N,), jnp.float32),
    mesh=_mesh,
    scratch_types=[
        pltpu.VMEM((NCH, CH), jnp.int32),
        pltpu.VMEM((NCH, CH), jnp.float32),
        pltpu.VMEM((NCH, CH), jnp.int32),
        pltpu.VMEM((_ROWS,), jnp.float32),
        pltpu.SemaphoreType.DMA,
        pltpu.VMEM_SHARED((N,), jnp.float32),
    ],
)


def _sc_agg_body(g_hbm, pk_hbm, w_hbm, out_hbm, *refs):
    pkv = refs[0:NBUF]
    wv = refs[NBUF:2 * NBUF]
    srcv = refs[2 * NBUF:3 * NBUF]
    dstv = refs[3 * NBUF:4 * NBUF]
    rows = refs[4 * NBUF:5 * NBUF]
    gsem = refs[5 * NBUF:6 * NBUF]
    isem = refs[6 * NBUF:7 * NBUF]
    ssem = refs[7 * NBUF:8 * NBUF]
    acc_sh = refs[8 * NBUF]

    cid = lax.axis_index("c")
    sid = lax.axis_index("s")
    roff = sid * _STRIDE
    ebase = (sid * NCHT + cid * NCH0) * CHA
    nch = jnp.where(cid == 0, NCH0, NCH1)

    def zrow(r, carry):
        for j in range(8):
            rows[0][r, pl.ds(j * 16, 16)] = jnp.zeros((16,), jnp.float32)
        return carry

    lax.fori_loop(0, CHA, zrow, 0)
    for t in range(_ROWS // 40):
        pltpu.sync_copy(rows[0].at[pl.ds(0, 40)],
                        acc_sh.at[pl.ds(roff + t * 40, 40)])
    plsc.subcore_barrier()

    def issue_idx(j, b):
        off = ebase + j * CHA
        pltpu.async_copy(pk_hbm.at[pl.ds(off, CHA)], pkv[b], isem[b])
        pltpu.async_copy(w_hbm.at[pl.ds(off, CHA)], wv[b], isem[b])

    def wait_idx(b):
        pltpu.make_async_copy(pk_hbm.at[pl.ds(0, CHA)], pkv[b], isem[b]).wait()
        pltpu.make_async_copy(w_hbm.at[pl.ds(0, CHA)], wv[b], isem[b]).wait()

    def unpack(b):
        for t in range(CHA // 16):
            sl = pl.ds(t * 16, 16)
            p16 = pkv[b][sl]
            dstv[b][sl] = jnp.bitwise_and(p16, _MASK)
            srcv[b][sl] = lax.shift_right_logical(p16, _SHIFT)

    def issue_gather(b):
        pltpu.async_copy(g_hbm.at[srcv[b]], rows[b], gsem[b])

    def wait_gather(b):
        pltpu.make_async_copy(g_hbm.at[pl.ds(0, CHA)], rows[b], gsem[b]).wait()

    def issue_scatter(b):
        pltpu.async_copy(rows[b], acc_sh.at[dstv[b]], ssem[b], add=True)

    def wait_scatter(b):
        pltpu.make_async_copy(rows[b], acc_sh.at[dstv[b]], ssem[b]).wait()

    def scale(b):
        def egroup(gi, c2):
            w16 = wv[b][pl.ds(gi * 16, 16)]
            for e in range(16):
                sp = jnp.broadcast_to(lax.slice(w16, (e,), (e + 1,)), (16,))
                r = gi * 16 + e
                for jj in range(8):
                    sl = pl.ds(jj * 16, 16)
                    rows[b][r, sl] = rows[b][r, sl] * sp
            return c2
        lax.fori_loop(0, CHA // 16, egroup, 0)

    for b in range(NBUF):
        issue_idx(b, b)
    for b in range(NBUF):
        wait_idx(b)
        unpack(b)
        issue_gather(b)

    def slot(j, b):
        wait_gather(b)
        scale(b)

        @pl.when(j + 2 < nch)
        def _():
            issue_idx(j + 2, b)

        pltpu.sync_copy(rows[b], acc_sh.at[dstv[b]], add=True)

        @pl.when(j + 2 < nch)
        def _():
            wait_idx(b)
            unpack(b)
            issue_gather(b)

    def pair(gp, carry):
        j0 = gp * NBUF
        for t in range(NBUF):
            slot(j0 + t, t)
        return carry

    lax.fori_loop(0, nch // NBUF, pair, 0)
    plsc.subcore_barrier()

    for t in range(_ROWS // 40):
        pltpu.sync_copy(acc_sh.at[pl.ds(roff + t * 40, 40)],
                        rows[0].at[pl.ds(0, 40)])
        pltpu.sync_copy(rows[0].at[pl.ds(0, 40)],
                        out_hbm.at[cid, pl.ds(roff + t * 40, 40)])


_sc_agg = pl.kernel(
    _sc_agg_body,
    out_type=jax.ShapeDtypeStruct((NC, N, D), jnp.float32),
    mesh=_mesh,
    scratch_types=(
        [pltpu.VMEM((CHA,), jnp.int32) for _ in range(NBUF)] +
        [pltpu.VMEM((CHA,), jnp.float32) for _ in range(NBUF)] +
        [pltpu.VMEM((CHA,), jnp.int32) for _ in range(NBUF)] +
        [pltpu.VMEM((CHA,), jnp.int32) for _ in range(NBUF)] +
        [pltpu.VMEM((CHA, D), jnp.float32) for _ in range(NBUF)] +
        [pltpu.SemaphoreType.DMA for _ in range(3 * NBUF)] +
        [pltpu.VMEM_SHARED((N, D), jnp.float32)]
    ),
)


def _tc_pre_body(x_ref, w_ref, deg_ref, g_ref, dis_ref):
    h = jnp.dot(x_ref[...], w_ref[...], preferred_element_type=jnp.float32)
    deg = deg_ref[0, :, 0] + deg_ref[1, :, 0] + 1.0
    dis = lax.rsqrt(deg)
    g_ref[...] = h * dis[:, None]
    dis_ref[...] = dis[:, None]


_BR = 1000

_tc_pre = pl.pallas_call(
    _tc_pre_body,
    grid=(N // _BR,),
    in_specs=[
        pl.BlockSpec((_BR, D), lambda i: (i, 0)),
        pl.BlockSpec((D, D), lambda i: (0, 0)),
        pl.BlockSpec((NC, _BR, 1), lambda i: (0, i, 0)),
    ],
    out_specs=[
        pl.BlockSpec((_BR, D), lambda i: (i, 0)),
        pl.BlockSpec((_BR, 1), lambda i: (i, 0)),
    ],
    out_shape=[
        jax.ShapeDtypeStruct((N, D), jnp.float32),
        jax.ShapeDtypeStruct((N, 1), jnp.float32),
    ],
)


def _tc_post_body(acc_ref, g_ref, dis_ref, b_ref, o_ref):
    s = acc_ref[0] + acc_ref[1] + g_ref[...]
    o_ref[...] = jnp.maximum(s * dis_ref[...] + b_ref[...], 0.0)


_tc_post = pl.pallas_call(
    _tc_post_body,
    grid=(N // _BR,),
    in_specs=[
        pl.BlockSpec((NC, _BR, D), lambda i: (0, i, 0)),
        pl.BlockSpec((_BR, D), lambda i: (i, 0)),
        pl.BlockSpec((_BR, 1), lambda i: (i, 0)),
        pl.BlockSpec((1, D), lambda i: (0, 0)),
    ],
    out_specs=pl.BlockSpec((_BR, D), lambda i: (i, 0)),
    out_shape=jax.ShapeDtypeStruct((N, D), jnp.float32),
)


def kernel(x, edge_index, edge_weights, W, b):
    src = edge_index[0]
    dst = edge_index[1]
    packed = src * (1 << _SHIFT) + dst
    pad2 = lambda a: jnp.pad(a, (0, EPAD - E)).reshape(NW * NCH, CH)
    pad1 = lambda a: jnp.pad(a, (0, EPADA - E))
    deg2 = _sc_deg(pad2(packed), pad2(edge_weights))
    g, dis = _tc_pre(x, W, deg2.reshape(NC, N, 1))
    acc2 = _sc_agg(g, pad1(packed), pad1(edge_weights))
    return _tc_post(acc2, g, dis, b.reshape(1, D))


# speedup vs baseline: 1.4755x; 1.2210x over previous
import jax
import jax.numpy as jnp
from jax import lax
from jax.experimental import pallas as pl
from jax.experimental.pallas import tpu as pltpu
from jax.experimental.pallas import tpu_sc as plsc

N = 10000
E = 320000
D = 128

NC = 2
NS = 16
NW = NC * NS

CH = 64
NCH = 160
EPW = NCH * CH
EPAD = NW * EPW

CHA = 64
NBUF = 2
NCH0 = 226
NCH1 = 94
NCHT = NCH0 + NCH1
EPADA = NS * NCHT * CHA

_SHIFT = 14
_MASK = (1 << _SHIFT) - 1

_STRIDE = 624
_ROWS = 640

_mesh = plsc.VectorSubcoreMesh(core_axis_name="c", subcore_axis_name="s")


def _sc_deg_body(pk_hbm, w_hbm, out_hbm, pm, wm, dstm, dbuf, sem, deg_sh):
    cid = lax.axis_index("c")
    sid = lax.axis_index("s")
    wid = sid * NC + cid
    roff = sid * _STRIDE

    for t in range(_ROWS // 16):
        dbuf[pl.ds(t * 16, 16)] = jnp.zeros((16,), jnp.float32)
    pltpu.sync_copy(dbuf, deg_sh.at[pl.ds(roff, _ROWS)])

    pltpu.sync_copy(pk_hbm.at[pl.ds(wid * NCH, NCH)], pm)
    pltpu.sync_copy(w_hbm.at[pl.ds(wid * NCH, NCH)], wm)

    def unpack(j, carry):
        for t in range(CH // 16):
            sl = pl.ds(t * 16, 16)
            dstm[j, sl] = jnp.bitwise_and(pm[j, sl], _MASK)
        return carry

    lax.fori_loop(0, NCH, unpack, 0)
    plsc.subcore_barrier()

    def group(gg, carry):
        for t in range(8):
            j = gg * 8 + t
            pltpu.async_copy(wm.at[j], deg_sh.at[dstm.at[j]], sem, add=True)
        for t in range(8):
            j = gg * 8 + t
            pltpu.make_async_copy(wm.at[j], deg_sh.at[dstm.at[j]], sem).wait()
        return carry

    lax.fori_loop(0, NCH // 8, group, 0)
    plsc.subcore_barrier()

    pltpu.sync_copy(deg_sh.at[pl.ds(roff, _ROWS)], dbuf)
    pltpu.sync_copy(dbuf, out_hbm.at[pl.ds(cid * N + roff, _ROWS)])


_sc_deg = pl.kernel(
    _sc_deg_body,
    out_type=jax.ShapeDtypeStruct((NC * N,), jnp.float32),
    mesh=_mesh,
    scratch_types=[
        pltpu.VMEM((NCH, CH), jnp.int32),
        pltpu.VMEM((NCH, CH), jnp.float32),
        pltpu.VMEM((NCH, CH), jnp.int32),
        pltpu.VMEM((_ROWS,), jnp.float32),
        pltpu.SemaphoreType.DMA,
        pltpu.VMEM_SHARED((N,), jnp.float32),
    ],
)


def _sc_agg_body(g_hbm, pk_hbm, w_hbm, out_hbm, *refs):
    pkv = refs[0:NBUF]
    wv = refs[NBUF:2 * NBUF]
    srcv = refs[2 * NBUF:3 * NBUF]
    dstv = refs[3 * NBUF:4 * NBUF]
    rows = refs[4 * NBUF:5 * NBUF]
    gsem = refs[5 * NBUF:6 * NBUF]
    isem = refs[6 * NBUF:7 * NBUF]
    ssem = refs[7 * NBUF:8 * NBUF]
    acc_sh = refs[8 * NBUF]

    cid = lax.axis_index("c")
    sid = lax.axis_index("s")
    roff = sid * _STRIDE
    ebase = (sid * NCHT + cid * NCH0) * CHA
    nch = jnp.where(cid == 0, NCH0, NCH1)

    def zrow(r, carry):
        for j in range(8):
            rows[0][r, pl.ds(j * 16, 16)] = jnp.zeros((16,), jnp.float32)
        return carry

    lax.fori_loop(0, CHA, zrow, 0)
    for t in range(_ROWS // 40):
        pltpu.sync_copy(rows[0].at[pl.ds(0, 40)],
                        acc_sh.at[pl.ds(roff + t * 40, 40)])
    plsc.subcore_barrier()

    def issue_idx(j, b):
        off = ebase + j * CHA
        pltpu.async_copy(pk_hbm.at[pl.ds(off, CHA)], pkv[b], isem[b])
        pltpu.async_copy(w_hbm.at[pl.ds(off, CHA)], wv[b], isem[b])

    def wait_idx(b):
        pltpu.make_async_copy(pk_hbm.at[pl.ds(0, CHA)], pkv[b], isem[b]).wait()
        pltpu.make_async_copy(w_hbm.at[pl.ds(0, CHA)], wv[b], isem[b]).wait()

    def unpack(b):
        for t in range(CHA // 16):
            sl = pl.ds(t * 16, 16)
            p16 = pkv[b][sl]
            dstv[b][sl] = jnp.bitwise_and(p16, _MASK)
            srcv[b][sl] = lax.shift_right_logical(p16, _SHIFT)

    def issue_gather(b):
        pltpu.async_copy(g_hbm.at[srcv[b]], rows[b], gsem[b])

    def wait_gather(b):
        pltpu.make_async_copy(g_hbm.at[pl.ds(0, CHA)], rows[b], gsem[b]).wait()

    def issue_scatter(b):
        pltpu.async_copy(rows[b], acc_sh.at[dstv[b]], ssem[b], add=True)

    def wait_scatter(b):
        pltpu.make_async_copy(rows[b], acc_sh.at[dstv[b]], ssem[b]).wait()

    def scale(b):
        def egroup(gi, c2):
            w16 = wv[b][pl.ds(gi * 16, 16)]
            for e in range(16):
                sp = jnp.broadcast_to(lax.slice(w16, (e,), (e + 1,)), (16,))
                r = gi * 16 + e
                for jj in range(8):
                    sl = pl.ds(jj * 16, 16)
                    rows[b][r, sl] = rows[b][r, sl] * sp
            return c2
        lax.fori_loop(0, CHA // 16, egroup, 0)

    for b in range(NBUF):
        issue_idx(b, b)
    for b in range(NBUF):
        wait_idx(b)
        unpack(b)
        issue_gather(b)

    def slot(j, b):
        wait_gather(b)
        scale(b)

        @pl.when(j + 2 < nch)
        def _():
            issue_idx(j + 2, b)

        pltpu.sync_copy(rows[b], acc_sh.at[dstv[b]], add=True)

        @pl.when(j + 2 < nch)
        def _():
            wait_idx(b)
            unpack(b)
            issue_gather(b)

    def pair(gp, carry):
        j0 = gp * NBUF
        for t in range(NBUF):
            slot(j0 + t, t)
        return carry

    lax.fori_loop(0, nch // NBUF, pair, 0)
    plsc.subcore_barrier()

    for t in range(_ROWS // 40):
        pltpu.sync_copy(acc_sh.at[pl.ds(roff + t * 40, 40)],
                        rows[0].at[pl.ds(0, 40)])
        pltpu.sync_copy(rows[0].at[pl.ds(0, 40)],
                        out_hbm.at[cid, pl.ds(roff + t * 40, 40)])


_sc_agg = pl.kernel(
    _sc_agg_body,
    out_type=jax.ShapeDtypeStruct((NC, N, D), jnp.float32),
    mesh=_mesh,
    scratch_types=(
        [pltpu.VMEM((CHA,), jnp.int32) for _ in range(NBUF)] +
        [pltpu.VMEM((CHA,), jnp.float32) for _ in range(NBUF)] +
        [pltpu.VMEM((CHA,), jnp.int32) for _ in range(NBUF)] +
        [pltpu.VMEM((CHA,), jnp.int32) for _ in range(NBUF)] +
        [pltpu.VMEM((CHA, D), jnp.float32) for _ in range(NBUF)] +
        [pltpu.SemaphoreType.DMA for _ in range(3 * NBUF)] +
        [pltpu.VMEM_SHARED((N, D), jnp.float32)]
    ),
)


def _tc_pre_body(x_ref, w_ref, deg_ref, g_ref, dis_ref):
    h = jnp.dot(x_ref[...], w_ref[...], preferred_element_type=jnp.float32)
    deg = deg_ref[0, :, 0] + deg_ref[1, :, 0] + 1.0
    dis = lax.rsqrt(deg)
    g_ref[...] = h * dis[:, None]
    dis_ref[...] = dis[:, None]


_BR = 1000

_tc_pre = pl.pallas_call(
    _tc_pre_body,
    grid=(N // _BR,),
    in_specs=[
        pl.BlockSpec((_BR, D), lambda i: (i, 0)),
        pl.BlockSpec((D, D), lambda i: (0, 0)),
        pl.BlockSpec((NC, _BR, 1), lambda i: (0, i, 0)),
    ],
    out_specs=[
        pl.BlockSpec((_BR, D), lambda i: (i, 0)),
        pl.BlockSpec((_BR, 1), lambda i: (i, 0)),
    ],
    out_shape=[
        jax.ShapeDtypeStruct((N, D), jnp.float32),
        jax.ShapeDtypeStruct((N, 1), jnp.float32),
    ],
)


def _tc_post_body(acc_ref, g_ref, dis_ref, b_ref, o_ref):
    s = acc_ref[0] + acc_ref[1] + g_ref[...]
    o_ref[...] = jnp.maximum(s * dis_ref[...] + b_ref[...], 0.0)


_tc_post = pl.pallas_call(
    _tc_post_body,
    grid=(N // _BR,),
    in_specs=[
        pl.BlockSpec((NC, _BR, D), lambda i: (0, i, 0)),
        pl.BlockSpec((_BR, D), lambda i: (i, 0)),
        pl.BlockSpec((_BR, 1), lambda i: (i, 0)),
        pl.BlockSpec((1, D), lambda i: (0, 0)),
    ],
    out_specs=pl.BlockSpec((_BR, D), lambda i: (i, 0)),
    out_shape=jax.ShapeDtypeStruct((N, D), jnp.float32),
)


def kernel(x, edge_index, edge_weights, W, b):
    src = edge_index[0]
    dst = edge_index[1]
    packed = src * (1 << _SHIFT) + dst
    pad2 = lambda a: jnp.pad(a, (0, EPAD - E)).reshape(NW * NCH, CH)
    pad1 = lambda a: jnp.pad(a, (0, EPADA - E))
    deg2 = _sc_deg(pad2(packed), pad2(edge_weights))
    g, dis = _tc_pre(x, W, deg2.reshape(NC, N, 1))
    acc2 = _sc_agg(g, pad1(packed), pad1(edge_weights))
    return _tc_post(acc2, g, dis, b.reshape(1, D))
